# baseline (speedup 1.0000x reference)
import dataclasses
import functools

import jax
import jax.numpy as jnp
from jax import lax
from jax.experimental import pallas as pl
from jax.experimental.pallas import tpu as pltpu
from jax.experimental.pallas import tpu_sc as plsc

B = 4
LQ = 1024
DIM = 768
HEADS = 12
DH = 64
L = 3
P = 4
SHAPES = ((64, 64), (32, 32), (16, 16))
LVL_START = (0, 4096, 5120)
LV = 5376
K = L * P * 4
E = HEADS * K
N = B * LQ * HEADS
NW = 32
QPW = B * LQ // NW
OGRP = 8
GSLICE = 96
QBLK = 256
MBLK = 512


def _ln_mm_body(x_ref, w_ref, b_ref, wn_ref, bn_ref, o_ref):
    x = x_ref[...]
    m = jnp.mean(x, axis=-1, keepdims=True)
    v = jnp.mean((x - m) ** 2, axis=-1, keepdims=True)
    xn = (x - m) * lax.rsqrt(v + 1e-6) * wn_ref[...] + bn_ref[...]
    y = jnp.dot(xn.astype(jnp.bfloat16), w_ref[...],
                preferred_element_type=jnp.float32)
    o_ref[...] = (y + b_ref[...]).astype(jnp.bfloat16)


def _ln_matmul(x, w_bf, b, wn, bn):
    m, nout = x.shape[0], w_bf.shape[1]
    return pl.pallas_call(
        _ln_mm_body,
        grid=(m // MBLK,),
        in_specs=[
            pl.BlockSpec((MBLK, DIM), lambda i: (i, 0)),
            pl.BlockSpec((DIM, nout), lambda i: (0, 0)),
            pl.BlockSpec((nout,), lambda i: (0,)),
            pl.BlockSpec((DIM,), lambda i: (0,)),
            pl.BlockSpec((DIM,), lambda i: (0,)),
        ],
        out_specs=pl.BlockSpec((MBLK, nout), lambda i: (i, 0)),
        out_shape=jax.ShapeDtypeStruct((m, nout), jnp.bfloat16),
    )(x, w_bf, b, wn, bn)


LHP = L * HEADS * P


def _samp_body(q_ref, rp_ref, woff_ref, boff_ref, wa_ref, ba_ref,
               wn_ref, bn_ref, idx_ref, wts_ref):
    i = pl.program_id(0)
    b = i // (LQ // QBLK)
    x = q_ref[...]
    m = jnp.mean(x, axis=-1, keepdims=True)
    v = jnp.mean((x - m) ** 2, axis=-1, keepdims=True)
    qn = (x - m) * lax.rsqrt(v + 1e-6) * wn_ref[...] + bn_ref[...]
    qb = qn.astype(jnp.bfloat16)
    off = jnp.dot(qb, woff_ref[...], preferred_element_type=jnp.float32)
    off = off + boff_ref[...]
    logits = jnp.dot(qb, wa_ref[...], preferred_element_type=jnp.float32)
    logits = logits + ba_ref[...]
    e = jnp.exp(logits)
    gh = (lax.broadcasted_iota(jnp.int32, (LHP, HEADS), 0)
          % (HEADS * P)) // P
    g = (gh == lax.broadcasted_iota(jnp.int32, (LHP, HEADS), 1)
         ).astype(jnp.float32)
    denom = jax.lax.dot(e, g, precision=jax.lax.Precision.HIGHEST)
    denom_e = jax.lax.dot(denom, g.T, precision=jax.lax.Precision.HIGHEST)
    aw = e / denom_e
    lane = lax.iota(jnp.int32, LHP)
    lvl = lane // (HEADS * P)
    hlane = (lane - lvl * (HEADS * P)) // P
    wf = jnp.where(lvl == 0, float(SHAPES[0][1]),
                   jnp.where(lvl == 1, float(SHAPES[1][1]),
                             float(SHAPES[2][1])))
    hf = jnp.where(lvl == 0, float(SHAPES[0][0]),
                   jnp.where(lvl == 1, float(SHAPES[1][0]),
                             float(SHAPES[2][0])))
    wli = jnp.where(lvl == 0, SHAPES[0][1],
                    jnp.where(lvl == 1, SHAPES[1][1], SHAPES[2][1]))
    start = jnp.where(lvl == 0, LVL_START[0],
                      jnp.where(lvl == 1, LVL_START[1], LVL_START[2]))
    rx = jnp.concatenate(
        [jnp.broadcast_to(rp_ref[:, l, 0][:, None], (QBLK, HEADS * P))
         for l in range(L)], axis=1)
    ry = jnp.concatenate(
        [jnp.broadcast_to(rp_ref[:, l, 1][:, None], (QBLK, HEADS * P))
         for l in range(L)], axis=1)
    offx = off[:, :LHP]
    offy = off[:, LHP:]
    sx = (rx + offx / wf) * wf - 0.5
    sy = (ry + offy / hf) * hf - 0.5
    x0 = jnp.floor(sx)
    y0 = jnp.floor(sy)
    wx1 = sx - x0
    wy1 = sy - y0
    wlim = wf - 1.0
    hlim = hf - 1.0
    base_b = b * LV
    for c in range(4):
        cx, cy = c & 1, c >> 1
        xi = x0 + cx
        yi = y0 + cy
        valid = ((xi >= 0) & (xi <= wlim) & (yi >= 0) & (yi <= hlim))
        wc = (wx1 if cx else (1.0 - wx1)) * (wy1 if cy else (1.0 - wy1))
        w = jnp.where(valid, wc, 0.0) * aw
        xc = jnp.clip(xi, 0.0, wlim).astype(jnp.int32)
        yc = jnp.clip(yi, 0.0, hlim).astype(jnp.int32)
        pos = start + yc * wli + xc
        row = (base_b + pos) * HEADS + hlane
        idx_ref[:, c * LHP:(c + 1) * LHP] = row
        wts_ref[:, c * LHP:(c + 1) * LHP] = w


def _sampling(q2, rp3, woff_bf, boff, wa_bf, ba, wn, bn):
    nq = q2.shape[0]
    grid = (nq // QBLK,)
    return pl.pallas_call(
        _samp_body,
        grid=grid,
        in_specs=[
            pl.BlockSpec((QBLK, DIM), lambda i: (i, 0)),
            pl.BlockSpec((QBLK, L, 2), lambda i: (i, 0, 0)),
            pl.BlockSpec((DIM, HEADS * L * P * 2), lambda i: (0, 0)),
            pl.BlockSpec((HEADS * L * P * 2,), lambda i: (0,)),
            pl.BlockSpec((DIM, HEADS * L * P), lambda i: (0, 0)),
            pl.BlockSpec((HEADS * L * P,), lambda i: (0,)),
            pl.BlockSpec((DIM,), lambda i: (0,)),
            pl.BlockSpec((DIM,), lambda i: (0,)),
        ],
        out_specs=[
            pl.BlockSpec((QBLK, E), lambda i: (i, 0)),
            pl.BlockSpec((QBLK, E), lambda i: (i, 0)),
        ],
        out_shape=[
            jax.ShapeDtypeStruct((nq, E), jnp.int32),
            jax.ShapeDtypeStruct((nq, E), jnp.float32),
        ],
    )(q2, rp3, woff_bf, boff, wa_bf, ba, wn, bn)


def _splat16(vec, k):
    idx = jnp.full((16, 1), k, dtype=jnp.int32)
    dnums = lax.GatherDimensionNumbers(
        offset_dims=(), collapsed_slice_dims=(0,), start_index_map=(0,))
    return lax.gather(vec, idx, dnums, (1,),
                      mode=lax.GatherScatterMode.PROMISE_IN_BOUNDS)


def _sc_gather_combine(value_rows, idx_flat, wts_flat, nq):
    qpw = nq // NW
    mesh = plsc.VectorSubcoreMesh(core_axis_name="c", subcore_axis_name="s")
    cp = pltpu.CompilerParams()
    if "needs_layout_passes" in pltpu.CompilerParams.__dataclass_fields__:
        cp = dataclasses.replace(cp, needs_layout_passes=False)
    if "use_tc_tiling_on_sc" in pltpu.CompilerParams.__dataclass_fields__:
        cp = dataclasses.replace(cp, use_tc_tiling_on_sc=False)

    @functools.partial(
        pl.kernel,
        out_type=jax.ShapeDtypeStruct((nq * HEADS, DH), jnp.bfloat16),
        mesh=mesh,
        compiler_params=cp,
        scratch_types=[
            pltpu.VMEM((4 * E,), jnp.int32),
            pltpu.VMEM((4 * E,), jnp.float32),
            pltpu.VMEM((2 * E, DH), jnp.bfloat16),
            pltpu.VMEM((OGRP * HEADS, DH), jnp.bfloat16),
            pltpu.SemaphoreType.DMA,
            pltpu.SemaphoreType.DMA,
            pltpu.SemaphoreType.DMA,
        ],
    )
    def k(value_hbm, idx_hbm, wts_hbm, out_hbm, idx_v, wts_v, rows_v,
          out_v, sem0, sem1, isem):
        sems = (sem0, sem1)
        wid = lax.axis_index("c") * 16 + lax.axis_index("s")
        q0 = wid * qpw
        lane16 = lax.iota(jnp.int32, 16)
        wpat = (lane16 // P) * (HEADS * P) + (lane16 - (lane16 // P) * P)

        def copy_pair(pt, slot, sync):
            gb = (q0 + 2 * pt) * E
            pairs = ((idx_hbm, idx_v), (wts_hbm, wts_v))
            for src, dst in pairs:
                s = src.at[pl.ds(gb, 2 * E)]
                d = dst.at[pl.ds(slot * (2 * E), 2 * E)]
                if sync:
                    pltpu.sync_copy(s, d)
                else:
                    pltpu.async_copy(s, d, isem)

        def wait_pair(slot):
            for src, dst in ((idx_hbm, idx_v), (wts_hbm, wts_v)):
                pltpu.make_async_copy(
                    src.at[pl.ds(0, 2 * E)],
                    dst.at[pl.ds(slot * (2 * E), 2 * E)], isem).wait()

        def fire_rows(slot, half, rbuf):
            ibase = slot * (2 * E) + half * E
            for j in range(E // GSLICE):
                pltpu.async_copy(
                    value_hbm.at[idx_v.at[pl.ds(ibase + j * GSLICE,
                                                GSLICE)]],
                    rows_v.at[pl.ds(rbuf * E + j * GSLICE, GSLICE)],
                    sems[rbuf])

        def drain_rows(rbuf):
            pltpu.make_async_copy(value_hbm.at[pl.ds(0, E)],
                                  rows_v.at[pl.ds(rbuf * E, E)],
                                  sems[rbuf]).wait()

        def compute(qi_slot, rbuf, wbase):
            @pl.loop(0, HEADS)
            def _one(h):
                eb_r = rbuf * E + h * P
                eb_w = wbase + h * P
                acc = [jnp.zeros((32,), jnp.bfloat16)
                       for _ in range(DH // 32)]
                for g in range(3):
                    wvec = plsc.load_gather(
                        wts_v,
                        [lax.broadcast(eb_w + g * 4 * (HEADS * P), (16,))
                         + wpat])
                    for kk in range(16):
                        wt = _splat16(wvec, kk)
                        wtb = plsc.pack(wt, wt,
                                        format=plsc.PackFormat.INTERLEAVED)
                        ent = (eb_r + (g * 4 + kk // P) * (HEADS * P)
                               + kk % P)
                        for d in range(DH // 32):
                            acc[d] = acc[d] + wtb * rows_v[ent,
                                                           pl.ds(d * 32, 32)]
                orow = qi_slot * HEADS + h
                for d in range(DH // 32):
                    out_v[orow, pl.ds(d * 32, 32)] = acc[d]

        T = qpw // 2
        copy_pair(0, 0, sync=True)
        fire_rows(0, 0, 0)
        fire_rows(0, 1, 1)
        copy_pair(1, 1, sync=False)

        @pl.loop(0, T)
        def _pair(t):
            base = q0 + 2 * t
            ps = t - (t // 2) * 2
            ns = 1 - ps
            tm = t - (t // (OGRP // 2)) * (OGRP // 2)
            drain_rows(0)
            compute(2 * tm, 0, ps * (2 * E))

            @pl.when(t < T - 1)
            def _():
                wait_pair(ns)
                fire_rows(ns, 0, 0)

            drain_rows(1)
            compute(2 * tm + 1, 1, ps * (2 * E) + E)

            @pl.when(t < T - 1)
            def _():
                fire_rows(ns, 1, 1)

            @pl.when(t < T - 2)
            def _():
                copy_pair(t + 2, ps, sync=False)

            @pl.when(tm == OGRP // 2 - 1)
            def _():
                pltpu.sync_copy(
                    out_v,
                    out_hbm.at[pl.ds((base + 2 - OGRP) * HEADS,
                                     OGRP * HEADS)])

    return k(value_rows, idx_flat, wts_flat)


def _out_body(a_ref, w_ref, b_ref, q_ref, g_ref, o_ref):
    y = jnp.dot(a_ref[...].astype(jnp.bfloat16), w_ref[...],
                preferred_element_type=jnp.float32)
    o_ref[...] = q_ref[...] + g_ref[...] * (y + b_ref[...])


def _outproj(attn2, wout_bf, bout, q2, gamma):
    m = attn2.shape[0]
    return pl.pallas_call(
        _out_body,
        grid=(m // MBLK,),
        in_specs=[
            pl.BlockSpec((MBLK, DIM), lambda i: (i, 0)),
            pl.BlockSpec((DIM, DIM), lambda i: (0, 0)),
            pl.BlockSpec((DIM,), lambda i: (0,)),
            pl.BlockSpec((MBLK, DIM), lambda i: (i, 0)),
            pl.BlockSpec((DIM,), lambda i: (0,)),
        ],
        out_specs=pl.BlockSpec((MBLK, DIM), lambda i: (i, 0)),
        out_shape=jax.ShapeDtypeStruct((m, DIM), jnp.float32),
    )(attn2, wout_bf, bout, q2, gamma)


def kernel(query, reference_points, feat, spatial_shapes, lvl_start,
           qn_w, qn_b, fn_w, fn_b, Wv, bv, Woff, boff, Wa, ba,
           Wout, bout, gamma):
    wv_bf = Wv.astype(jnp.bfloat16)
    woff_bf = (Woff.reshape(DIM, HEADS, L, P, 2)
               .transpose(0, 4, 2, 1, 3).reshape(DIM, HEADS * L * P * 2)
               .astype(jnp.bfloat16))
    boff_p = (boff.reshape(HEADS, L, P, 2)
              .transpose(3, 1, 0, 2).reshape(HEADS * L * P * 2))
    wa_bf = (Wa.reshape(DIM, HEADS, L, P)
             .transpose(0, 2, 1, 3).reshape(DIM, HEADS * L * P)
             .astype(jnp.bfloat16))
    ba_p = ba.reshape(HEADS, L, P).transpose(1, 0, 2).reshape(HEADS * L * P)
    wout_bf = Wout.astype(jnp.bfloat16)

    halves = []
    hb = B // 2
    for h in range(2):
        qh = query[h * hb:(h + 1) * hb].reshape(hb * LQ, DIM)
        rph = reference_points[h * hb:(h + 1) * hb].reshape(hb * LQ, L, 2)
        fh = feat[h * hb:(h + 1) * hb].reshape(hb * LV, DIM)
        value = _ln_matmul(fh, wv_bf, bv, fn_w, fn_b)
        idx, wts = _sampling(qh, rph, woff_bf, boff_p, wa_bf, ba_p,
                             qn_w, qn_b)
        attn = _sc_gather_combine(value.reshape(hb * LV * HEADS, DH),
                                  idx.reshape(hb * LQ * E),
                                  wts.reshape(hb * LQ * E), hb * LQ)
        out = _outproj(attn.reshape(hb * LQ, DIM), wout_bf, bout,
                       qh, gamma)
        halves.append(out.reshape(hb, LQ, DIM))
    return jnp.concatenate(halves, axis=0)


# speedup vs baseline: 1.1115x; 1.1115x over previous
import dataclasses
import functools

import jax
import jax.numpy as jnp
from jax import lax
from jax.experimental import pallas as pl
from jax.experimental.pallas import tpu as pltpu
from jax.experimental.pallas import tpu_sc as plsc

B = 4
LQ = 1024
DIM = 768
HEADS = 12
DH = 64
L = 3
P = 4
SHAPES = ((64, 64), (32, 32), (16, 16))
LVL_START = (0, 4096, 5120)
LV = 5376
K = L * P * 4
E = HEADS * K
N = B * LQ * HEADS
NW = 32
QPW = B * LQ // NW
OGRP = 8
GS_SLICES = ((0, 128), (128, 128), (256, 128), (384, 128), (512, 64))
QBLK = 256
MBLK = 512


def _ln_mm_body(x_ref, w_ref, b_ref, wn_ref, bn_ref, o_ref):
    x = x_ref[...]
    m = jnp.mean(x, axis=-1, keepdims=True)
    v = jnp.mean((x - m) ** 2, axis=-1, keepdims=True)
    xn = (x - m) * lax.rsqrt(v + 1e-6) * wn_ref[...] + bn_ref[...]
    y = jnp.dot(xn.astype(jnp.bfloat16), w_ref[...],
                preferred_element_type=jnp.float32)
    o_ref[...] = (y + b_ref[...]).astype(jnp.bfloat16)


def _ln_matmul(x, w_bf, b, wn, bn):
    m, nout = x.shape[0], w_bf.shape[1]
    return pl.pallas_call(
        _ln_mm_body,
        grid=(m // MBLK,),
        in_specs=[
            pl.BlockSpec((MBLK, DIM), lambda i: (i, 0)),
            pl.BlockSpec((DIM, nout), lambda i: (0, 0)),
            pl.BlockSpec((nout,), lambda i: (0,)),
            pl.BlockSpec((DIM,), lambda i: (0,)),
            pl.BlockSpec((DIM,), lambda i: (0,)),
        ],
        out_specs=pl.BlockSpec((MBLK, nout), lambda i: (i, 0)),
        out_shape=jax.ShapeDtypeStruct((m, nout), jnp.bfloat16),
    )(x, w_bf, b, wn, bn)


LHP = L * HEADS * P


def _samp_body(q_ref, rp_ref, woff_ref, boff_ref, wa_ref, ba_ref,
               wn_ref, bn_ref, idx_ref, wts_ref):
    i = pl.program_id(0)
    b = i // (LQ // QBLK)
    x = q_ref[...]
    m = jnp.mean(x, axis=-1, keepdims=True)
    v = jnp.mean((x - m) ** 2, axis=-1, keepdims=True)
    qn = (x - m) * lax.rsqrt(v + 1e-6) * wn_ref[...] + bn_ref[...]
    qb = qn.astype(jnp.bfloat16)
    off = jnp.dot(qb, woff_ref[...], preferred_element_type=jnp.float32)
    off = off + boff_ref[...]
    logits = jnp.dot(qb, wa_ref[...], preferred_element_type=jnp.float32)
    logits = logits + ba_ref[...]
    e = jnp.exp(logits)
    gh = (lax.broadcasted_iota(jnp.int32, (LHP, HEADS), 0)
          % (HEADS * P)) // P
    g = (gh == lax.broadcasted_iota(jnp.int32, (LHP, HEADS), 1)
         ).astype(jnp.float32)
    denom = jax.lax.dot(e, g, precision=jax.lax.Precision.HIGHEST)
    denom_e = jax.lax.dot(denom, g.T, precision=jax.lax.Precision.HIGHEST)
    aw = e / denom_e
    lane = lax.iota(jnp.int32, LHP)
    lvl = lane // (HEADS * P)
    hlane = (lane - lvl * (HEADS * P)) // P
    wf = jnp.where(lvl == 0, float(SHAPES[0][1]),
                   jnp.where(lvl == 1, float(SHAPES[1][1]),
                             float(SHAPES[2][1])))
    hf = jnp.where(lvl == 0, float(SHAPES[0][0]),
                   jnp.where(lvl == 1, float(SHAPES[1][0]),
                             float(SHAPES[2][0])))
    wli = jnp.where(lvl == 0, SHAPES[0][1],
                    jnp.where(lvl == 1, SHAPES[1][1], SHAPES[2][1]))
    start = jnp.where(lvl == 0, LVL_START[0],
                      jnp.where(lvl == 1, LVL_START[1], LVL_START[2]))
    rx = jnp.concatenate(
        [jnp.broadcast_to(rp_ref[:, l, 0][:, None], (QBLK, HEADS * P))
         for l in range(L)], axis=1)
    ry = jnp.concatenate(
        [jnp.broadcast_to(rp_ref[:, l, 1][:, None], (QBLK, HEADS * P))
         for l in range(L)], axis=1)
    offx = off[:, :LHP]
    offy = off[:, LHP:]
    sx = (rx + offx / wf) * wf - 0.5
    sy = (ry + offy / hf) * hf - 0.5
    x0 = jnp.floor(sx)
    y0 = jnp.floor(sy)
    wx1 = sx - x0
    wy1 = sy - y0
    wlim = wf - 1.0
    hlim = hf - 1.0
    base_b = b * LV
    for c in range(4):
        cx, cy = c & 1, c >> 1
        xi = x0 + cx
        yi = y0 + cy
        valid = ((xi >= 0) & (xi <= wlim) & (yi >= 0) & (yi <= hlim))
        wc = (wx1 if cx else (1.0 - wx1)) * (wy1 if cy else (1.0 - wy1))
        w = jnp.where(valid, wc, 0.0) * aw
        xc = jnp.clip(xi, 0.0, wlim).astype(jnp.int32)
        yc = jnp.clip(yi, 0.0, hlim).astype(jnp.int32)
        pos = start + yc * wli + xc
        row = (base_b + pos) * HEADS + hlane
        idx_ref[:, c * LHP:(c + 1) * LHP] = row
        wts_ref[:, c * LHP:(c + 1) * LHP] = w


def _sampling(q2, rp3, woff_bf, boff, wa_bf, ba, wn, bn):
    nq = q2.shape[0]
    grid = (nq // QBLK,)
    return pl.pallas_call(
        _samp_body,
        grid=grid,
        in_specs=[
            pl.BlockSpec((QBLK, DIM), lambda i: (i, 0)),
            pl.BlockSpec((QBLK, L, 2), lambda i: (i, 0, 0)),
            pl.BlockSpec((DIM, HEADS * L * P * 2), lambda i: (0, 0)),
            pl.BlockSpec((HEADS * L * P * 2,), lambda i: (0,)),
            pl.BlockSpec((DIM, HEADS * L * P), lambda i: (0, 0)),
            pl.BlockSpec((HEADS * L * P,), lambda i: (0,)),
            pl.BlockSpec((DIM,), lambda i: (0,)),
            pl.BlockSpec((DIM,), lambda i: (0,)),
        ],
        out_specs=[
            pl.BlockSpec((QBLK, E), lambda i: (i, 0)),
            pl.BlockSpec((QBLK, E), lambda i: (i, 0)),
        ],
        out_shape=[
            jax.ShapeDtypeStruct((nq, E), jnp.int32),
            jax.ShapeDtypeStruct((nq, E), jnp.float32),
        ],
    )(q2, rp3, woff_bf, boff, wa_bf, ba, wn, bn)


def _splat16(vec, k):
    idx = jnp.full((16, 1), k, dtype=jnp.int32)
    dnums = lax.GatherDimensionNumbers(
        offset_dims=(), collapsed_slice_dims=(0,), start_index_map=(0,))
    return lax.gather(vec, idx, dnums, (1,),
                      mode=lax.GatherScatterMode.PROMISE_IN_BOUNDS)


def _sc_gather_combine(value_rows, idx_flat, wts_flat, nq):
    qpw = nq // NW
    mesh = plsc.VectorSubcoreMesh(core_axis_name="c", subcore_axis_name="s")
    cp = pltpu.CompilerParams()
    if "needs_layout_passes" in pltpu.CompilerParams.__dataclass_fields__:
        cp = dataclasses.replace(cp, needs_layout_passes=False)
    if "use_tc_tiling_on_sc" in pltpu.CompilerParams.__dataclass_fields__:
        cp = dataclasses.replace(cp, use_tc_tiling_on_sc=False)

    @functools.partial(
        pl.kernel,
        out_type=jax.ShapeDtypeStruct((nq * HEADS, DH), jnp.bfloat16),
        mesh=mesh,
        compiler_params=cp,
        scratch_types=[
            pltpu.VMEM((2 * 4 * E,), jnp.int32),
            pltpu.VMEM((2 * 4 * E,), jnp.float32),
            pltpu.VMEM((4 * E, DH), jnp.bfloat16),
            pltpu.VMEM((OGRP * HEADS, DH), jnp.bfloat16),
            pltpu.SemaphoreType.DMA,
            pltpu.SemaphoreType.DMA,
            pltpu.SemaphoreType.DMA,
            pltpu.SemaphoreType.DMA,
            pltpu.SemaphoreType.DMA,
        ],
    )
    def k(value_hbm, idx_hbm, wts_hbm, out_hbm, idx_v, wts_v, rows_v,
          out_v, sem0, sem1, sem2, sem3, isem):
        sems = (sem0, sem1, sem2, sem3)
        wid = lax.axis_index("c") * 16 + lax.axis_index("s")
        q0 = wid * qpw
        lane16 = lax.iota(jnp.int32, 16)
        wpat = (lane16 // P) * (HEADS * P) + (lane16 - (lane16 // P) * P)

        def copy_group(gt, slot, sync):
            gb = (q0 + 4 * gt) * E
            for src, dst in ((idx_hbm, idx_v), (wts_hbm, wts_v)):
                s = src.at[pl.ds(gb, 4 * E)]
                d = dst.at[pl.ds(slot * (4 * E), 4 * E)]
                if sync:
                    pltpu.sync_copy(s, d)
                else:
                    pltpu.async_copy(s, d, isem)

        def wait_group(slot):
            for src, dst in ((idx_hbm, idx_v), (wts_hbm, wts_v)):
                pltpu.make_async_copy(
                    src.at[pl.ds(0, 4 * E)],
                    dst.at[pl.ds(slot * (4 * E), 4 * E)], isem).wait()

        def fire_rows(slot, j, rbuf):
            ibase = slot * (4 * E) + j * E
            for o, n in GS_SLICES:
                pltpu.async_copy(
                    value_hbm.at[idx_v.at[pl.ds(ibase + o, n)]],
                    rows_v.at[pl.ds(rbuf * E + o, n)],
                    sems[rbuf])

        def drain_rows(rbuf):
            pltpu.make_async_copy(value_hbm.at[pl.ds(0, E)],
                                  rows_v.at[pl.ds(rbuf * E, E)],
                                  sems[rbuf]).wait()

        def compute(qi_slot, rbuf, wbase):
            @pl.loop(0, HEADS)
            def _one(h):
                eb_r = rbuf * E + h * P
                eb_w = wbase + h * P
                acc = [jnp.zeros((32,), jnp.bfloat16)
                       for _ in range(DH // 32)]
                for g in range(3):
                    wvec = plsc.load_gather(
                        wts_v,
                        [lax.broadcast(eb_w + g * 4 * (HEADS * P), (16,))
                         + wpat])
                    for kk in range(16):
                        wt = _splat16(wvec, kk)
                        wtb = plsc.pack(wt, wt,
                                        format=plsc.PackFormat.INTERLEAVED)
                        ent = (eb_r + (g * 4 + kk // P) * (HEADS * P)
                               + kk % P)
                        for d in range(DH // 32):
                            acc[d] = acc[d] + wtb * rows_v[ent,
                                                           pl.ds(d * 32, 32)]
                orow = qi_slot * HEADS + h
                for d in range(DH // 32):
                    out_v[orow, pl.ds(d * 32, 32)] = acc[d]

        ng = qpw // 4
        copy_group(0, 0, sync=True)
        for j in range(4):
            fire_rows(0, j, j)
        copy_group(1, 1, sync=False)

        @pl.loop(0, ng)
        def _grp(g):
            gm = g - (g // 2) * 2
            ns = 1 - gm

            @pl.when(g < ng - 1)
            def _():
                wait_group(ns)

            for j in range(4):
                drain_rows(j)
                compute(4 * gm + j, j, gm * (4 * E) + j * E)

                @pl.when(g < ng - 1)
                def _():
                    fire_rows(ns, j, j)

            @pl.when(g < ng - 2)
            def _():
                copy_group(g + 2, gm, sync=False)

            @pl.when(gm == 1)
            def _():
                pltpu.sync_copy(
                    out_v,
                    out_hbm.at[pl.ds((q0 + 4 * (g - 1)) * HEADS,
                                     OGRP * HEADS)])

    return k(value_rows, idx_flat, wts_flat)


def _out_body(a_ref, w_ref, b_ref, q_ref, g_ref, o_ref):
    y = jnp.dot(a_ref[...].astype(jnp.bfloat16), w_ref[...],
                preferred_element_type=jnp.float32)
    o_ref[...] = q_ref[...] + g_ref[...] * (y + b_ref[...])


def _outproj(attn2, wout_bf, bout, q2, gamma):
    m = attn2.shape[0]
    return pl.pallas_call(
        _out_body,
        grid=(m // MBLK,),
        in_specs=[
            pl.BlockSpec((MBLK, DIM), lambda i: (i, 0)),
            pl.BlockSpec((DIM, DIM), lambda i: (0, 0)),
            pl.BlockSpec((DIM,), lambda i: (0,)),
            pl.BlockSpec((MBLK, DIM), lambda i: (i, 0)),
            pl.BlockSpec((DIM,), lambda i: (0,)),
        ],
        out_specs=pl.BlockSpec((MBLK, DIM), lambda i: (i, 0)),
        out_shape=jax.ShapeDtypeStruct((m, DIM), jnp.float32),
    )(attn2, wout_bf, bout, q2, gamma)


def kernel(query, reference_points, feat, spatial_shapes, lvl_start,
           qn_w, qn_b, fn_w, fn_b, Wv, bv, Woff, boff, Wa, ba,
           Wout, bout, gamma):
    wv_bf = Wv.astype(jnp.bfloat16)
    woff_bf = (Woff.reshape(DIM, HEADS, L, P, 2)
               .transpose(0, 4, 2, 1, 3).reshape(DIM, HEADS * L * P * 2)
               .astype(jnp.bfloat16))
    boff_p = (boff.reshape(HEADS, L, P, 2)
              .transpose(3, 1, 0, 2).reshape(HEADS * L * P * 2))
    wa_bf = (Wa.reshape(DIM, HEADS, L, P)
             .transpose(0, 2, 1, 3).reshape(DIM, HEADS * L * P)
             .astype(jnp.bfloat16))
    ba_p = ba.reshape(HEADS, L, P).transpose(1, 0, 2).reshape(HEADS * L * P)
    wout_bf = Wout.astype(jnp.bfloat16)

    value = _ln_matmul(feat.reshape(B * LV, DIM), wv_bf, bv, fn_w, fn_b)
    idx, wts = _sampling(query.reshape(B * LQ, DIM),
                         reference_points.reshape(B * LQ, L, 2),
                         woff_bf, boff_p, wa_bf, ba_p, qn_w, qn_b)
    attn = _sc_gather_combine(value.reshape(B * LV * HEADS, DH),
                              idx.reshape(B * LQ * E),
                              wts.reshape(B * LQ * E), B * LQ)
    out = _outproj(attn.reshape(B * LQ, DIM), wout_bf, bout,
                   query.reshape(B * LQ, DIM), gamma)
    return out.reshape(B, LQ, DIM)
